# Initial kernel scaffold; baseline (speedup 1.0000x reference)
import jax
import jax.numpy as jnp
from jax.experimental import pallas as pl


def kernel(src, postag_ids, seg, word_table, pos_table, seg_table, postag_table, gamma, beta):
    raise NotImplementedError("write your pallas kernel here")


# speedup vs baseline: 2.1869x; 2.1869x over previous
import functools

import jax
import jax.numpy as jnp
from jax import lax
from jax.experimental import pallas as pl
from jax.experimental.pallas import tpu as pltpu
from jax.experimental.pallas import tpu_sc as plsc

_B = 256
_S = 512
_E = 128
_NSEG = 3
_EPS = 1e-6

_NC = 2
_NS = 16
_NW = _NC * _NS
_TOK = _B * _S
_PERW = _TOK // _NW
_C = 128
_NCHUNK = _PERW // _C
_L = 16
_NV = _E // _L


def _rsqrt16(x):
    i = lax.bitcast_convert_type(x, jnp.int32)
    i = jnp.int32(0x5F3759DF) - (i >> 1)
    y = lax.bitcast_convert_type(i, jnp.float32)
    half = x * 0.5
    for _ in range(3):
        y = y * (1.5 - half * y * y)
    return y


def _sc_body(wtab, pstab, pttab, widx, psidx, ptidx, gam, bet, out,
             widx_v, psidx_v, ptidx_v, gam_v, bet_v, bufw, bufps, bufpt,
             sem):
    wid = lax.axis_index("s") * _NC + lax.axis_index("c")

    pltpu.sync_copy(widx.at[wid], widx_v)
    pltpu.sync_copy(psidx.at[wid], psidx_v)
    pltpu.sync_copy(ptidx.at[wid], ptidx_v)
    pltpu.sync_copy(gam, gam_v)
    pltpu.sync_copy(bet, bet_v)

    base = wid * _PERW

    def chunk_body(c, carry):
        cp_w = pltpu.async_copy(wtab.at[widx_v.at[c]], bufw, sem)
        cp_ps = pltpu.async_copy(pstab.at[psidx_v.at[c]], bufps, sem)
        cp_pt = pltpu.async_copy(pttab.at[ptidx_v.at[c]], bufpt, sem)
        cp_w.wait()
        cp_ps.wait()
        cp_pt.wait()

        lanes = lax.iota(jnp.int32, _L)
        gdn = lax.GatherDimensionNumbers(
            offset_dims=(), collapsed_slice_dims=(0,), start_index_map=(0,))

        def shuffle(x, idx):
            return lax.gather(
                x, idx[:, None], gdn, slice_sizes=(1,),
                mode=lax.GatherScatterMode.PROMISE_IN_BOUNDS)

        def hsum(x):
            for k in (1, 2, 4, 8):
                x = x + shuffle(x, lanes ^ k)
            return x

        def tok_body(t, tcarry):
            rw = bufw.at[t]
            rps = bufps.at[t]
            rpt = bufpt.at[t]
            xs = []
            s1 = jnp.zeros((_L,), jnp.float32)
            s2 = jnp.zeros((_L,), jnp.float32)
            for v in range(_NV):
                sl = pl.ds(v * _L, _L)
                x = rw[sl] + rps[sl] + rpt[sl]
                xs.append(x)
                s1 = s1 + x
                s2 = s2 + x * x
            meanv = hsum(s1) * (1.0 / _E)
            varv = hsum(s2) * (1.0 / _E) - meanv * meanv
            inv = _rsqrt16(varv + _EPS)
            for v in range(_NV):
                sl = pl.ds(v * _L, _L)
                y = (xs[v] - meanv) * inv * gam_v[sl] + bet_v[sl]
                rw[sl] = y
            return tcarry

        lax.fori_loop(0, _C, tok_body, 0)
        pltpu.sync_copy(bufw, out.at[pl.ds(base + c * _C, _C)])
        return carry

    lax.fori_loop(0, _NCHUNK, chunk_body, 0)


def kernel(src, postag_ids, seg, word_table, pos_table, seg_table,
           postag_table, gamma, beta):
    ps_table = (pos_table[:, None, :] + seg_table[None, :, :]).reshape(
        _S * _NSEG, _E)

    src_i = src.astype(jnp.int32).reshape(_NW, _NCHUNK, _C)
    pos_ids = jnp.arange(_S, dtype=jnp.int32)
    ps_idx = (pos_ids[None, :] * _NSEG + seg.astype(jnp.int32)).reshape(
        _NW, _NCHUNK, _C)
    pt_idx = postag_ids.astype(jnp.int32).reshape(_NW, _NCHUNK, _C)

    mesh = plsc.VectorSubcoreMesh(core_axis_name="c", subcore_axis_name="s")
    run = functools.partial(
        pl.kernel,
        mesh=mesh,
        out_type=jax.ShapeDtypeStruct((_TOK, _E), jnp.float32),
        scratch_types=[
            pltpu.VMEM((_NCHUNK, _C), jnp.int32),
            pltpu.VMEM((_NCHUNK, _C), jnp.int32),
            pltpu.VMEM((_NCHUNK, _C), jnp.int32),
            pltpu.VMEM((_E,), jnp.float32),
            pltpu.VMEM((_E,), jnp.float32),
            pltpu.VMEM((_C, _E), jnp.float32),
            pltpu.VMEM((_C, _E), jnp.float32),
            pltpu.VMEM((_C, _E), jnp.float32),
            pltpu.SemaphoreType.DMA,
        ],
    )(_sc_body)
    out = run(word_table, ps_table, postag_table, src_i, ps_idx, pt_idx,
              gamma, beta)
    return out.reshape(_B, _S, _E)


# speedup vs baseline: 3.6446x; 1.6666x over previous
import functools

import jax
import jax.numpy as jnp
from jax import lax
from jax.experimental import pallas as pl
from jax.experimental.pallas import tpu as pltpu
from jax.experimental.pallas import tpu_sc as plsc

_B = 256
_S = 512
_E = 128
_NSEG = 3
_EPS = 1e-6

_NC = 2
_NS = 16
_NW = _NC * _NS
_TOK = _B * _S
_PERW = _TOK // _NW
_C = 128
_NCHUNK = _PERW // _C
_L = 16
_NV = _E // _L


def _rsqrt16(x):
    i = lax.bitcast_convert_type(x, jnp.int32)
    i = jnp.int32(0x5F3759DF) - (i >> 1)
    y = lax.bitcast_convert_type(i, jnp.float32)
    half = x * 0.5
    for _ in range(3):
        y = y * (1.5 - half * y * y)
    return y


def _sc_body(wtab, pstab, pttab, widx, psidx, ptidx, gam, bet, out,
             widx_v, psidx_v, ptidx_v, gam_v, bet_v,
             bufw0, bufps0, bufpt0, bufw1, bufps1, bufpt1, sem0, sem1):
    wid = lax.axis_index("s") * _NC + lax.axis_index("c")

    pltpu.sync_copy(widx.at[wid], widx_v)
    pltpu.sync_copy(psidx.at[wid], psidx_v)
    pltpu.sync_copy(ptidx.at[wid], ptidx_v)
    pltpu.sync_copy(gam, gam_v)
    pltpu.sync_copy(bet, bet_v)

    base = wid * _PERW
    bufs = ((bufw0, bufps0, bufpt0, sem0), (bufw1, bufps1, bufpt1, sem1))

    def fire(k, b):
        bw, bp, bt, sem = bufs[b]
        pltpu.async_copy(wtab.at[widx_v.at[k]], bw, sem)
        pltpu.async_copy(pstab.at[psidx_v.at[k]], bp, sem)
        pltpu.async_copy(pttab.at[ptidx_v.at[k]], bt, sem)

    def drain(k, b):
        bw, bp, bt, sem = bufs[b]
        pltpu.make_async_copy(wtab.at[widx_v.at[k]], bw, sem).wait()
        pltpu.make_async_copy(pstab.at[psidx_v.at[k]], bp, sem).wait()
        pltpu.make_async_copy(pttab.at[ptidx_v.at[k]], bt, sem).wait()

    lanes = lax.iota(jnp.int32, _L)
    gdn = lax.GatherDimensionNumbers(
        offset_dims=(), collapsed_slice_dims=(0,), start_index_map=(0,))

    def shuffle(x, idx):
        return lax.gather(
            x, idx[:, None], gdn, slice_sizes=(1,),
            mode=lax.GatherScatterMode.PROMISE_IN_BOUNDS)

    def hsum(x):
        for k in (1, 2, 4, 8):
            x = x + shuffle(x, lanes ^ k)
        return x

    def compute_chunk(k, b):
        bw, bp, bt, _ = bufs[b]

        @plsc.parallel_loop(0, _C, unroll=2)
        def tok_body(t):
            rw = bw.at[t]
            rps = bp.at[t]
            rpt = bt.at[t]
            xs = []
            s1 = jnp.zeros((_L,), jnp.float32)
            s2 = jnp.zeros((_L,), jnp.float32)
            for v in range(_NV):
                sl = pl.ds(v * _L, _L)
                x = rw[sl] + rps[sl] + rpt[sl]
                xs.append(x)
                s1 = s1 + x
                s2 = s2 + x * x
            meanv = hsum(s1) * (1.0 / _E)
            varv = hsum(s2) * (1.0 / _E) - meanv * meanv
            inv = _rsqrt16(varv + _EPS)
            for v in range(_NV):
                sl = pl.ds(v * _L, _L)
                y = (xs[v] - meanv) * inv * gam_v[sl] + bet_v[sl]
                rw[sl] = y

        pltpu.sync_copy(bw, out.at[pl.ds(base + k * _C, _C)])

    fire(0, 0)

    @pl.loop(0, _NCHUNK, step=2)
    def chunk_pair(c):
        for b in (0, 1):
            k = c + b
            nxt = k + 1
            if b == 0:
                fire(nxt, 1)
            else:
                @pl.when(nxt < _NCHUNK)
                def _():
                    fire(nxt, 0)
            drain(k, b)
            compute_chunk(k, b)


def kernel(src, postag_ids, seg, word_table, pos_table, seg_table,
           postag_table, gamma, beta):
    ps_table = (pos_table[:, None, :] + seg_table[None, :, :]).reshape(
        _S * _NSEG, _E)

    src_i = src.astype(jnp.int32).reshape(_NW, _NCHUNK, _C)
    pos_ids = jnp.arange(_S, dtype=jnp.int32)
    ps_idx = (pos_ids[None, :] * _NSEG + seg.astype(jnp.int32)).reshape(
        _NW, _NCHUNK, _C)
    pt_idx = postag_ids.astype(jnp.int32).reshape(_NW, _NCHUNK, _C)

    mesh = plsc.VectorSubcoreMesh(core_axis_name="c", subcore_axis_name="s")
    run = functools.partial(
        pl.kernel,
        mesh=mesh,
        out_type=jax.ShapeDtypeStruct((_TOK, _E), jnp.float32),
        scratch_types=[
            pltpu.VMEM((_NCHUNK, _C), jnp.int32),
            pltpu.VMEM((_NCHUNK, _C), jnp.int32),
            pltpu.VMEM((_NCHUNK, _C), jnp.int32),
            pltpu.VMEM((_E,), jnp.float32),
            pltpu.VMEM((_E,), jnp.float32),
            pltpu.VMEM((_C, _E), jnp.float32),
            pltpu.VMEM((_C, _E), jnp.float32),
            pltpu.VMEM((_C, _E), jnp.float32),
            pltpu.VMEM((_C, _E), jnp.float32),
            pltpu.VMEM((_C, _E), jnp.float32),
            pltpu.VMEM((_C, _E), jnp.float32),
            pltpu.SemaphoreType.DMA,
            pltpu.SemaphoreType.DMA,
        ],
    )(_sc_body)
    out = run(word_table, ps_table, postag_table, src_i, ps_idx, pt_idx,
              gamma, beta)
    return out.reshape(_B, _S, _E)
